# baseline (speedup 1.0000x reference)
import jax
import jax.numpy as jnp
from jax import lax
from jax.experimental import pallas as pl
from jax.experimental.pallas import tpu as pltpu
from jax.experimental.pallas import tpu_sc as plsc

B = 1024
D = 64
_NC = 2
_NS = 16
_NW = _NC * _NS
_BPW = B // _NW


_RPR = 4
_NBUF = 3
_NROUND = 2 * _BPW // _RPR


def _gather_body(u_idx, i_idx, u_tabT, i_tabT, u_out, i_out,
                 uidx_v, iidx_v, urows_v, irows_v, slabs, sems):
    wid = lax.axis_index("s") * _NC + lax.axis_index("c")
    base = wid * _BPW
    pltpu.sync_copy(u_idx.at[pl.ds(base, _BPW)], uidx_v)
    pltpu.sync_copy(i_idx.at[pl.ds(base, _BPW)], iidx_v)

    idx_vecs = []
    for k in range(_BPW // 16):
        idx_vecs.append(uidx_v[pl.ds(16 * k, 16)])
    for k in range(_BPW // 16):
        idx_vecs.append(iidx_v[pl.ds(16 * k, 16)])

    def row_scalar(j):
        return idx_vecs[j // 16][j % 16]

    def fire(rnd, buf):
        tab = u_tabT if rnd < _NROUND // 2 else i_tabT
        cps = []
        for s in range(_RPR):
            r = row_scalar(rnd * _RPR + s)
            col0 = pl.multiple_of((r >> 7) << 7, 128)
            cps.append(pltpu.async_copy(
                tab.at[:, pl.ds(col0, 128)], slabs.at[buf * _RPR + s],
                sems.at[buf]))
        return cps

    lanes = lax.iota(jnp.int32, 16)
    pend = {0: fire(0, 0), 1: fire(1, 1)}
    for rnd in range(_NROUND):
        buf = rnd % _NBUF
        if rnd + 2 < _NROUND:
            pend[(rnd + 2) % _NBUF] = fire(rnd + 2, (rnd + 2) % _NBUF)
        for cp in pend.pop(buf):
            cp.wait()
        for s in range(_RPR):
            j = rnd * _RPR + s
            r = row_scalar(j)
            loc = jnp.broadcast_to(r & 127, (16,))
            dst = urows_v if j < _BPW else irows_v
            jj = j % _BPW
            for a in range(D // 16):
                vals = plsc.load_gather(
                    slabs.at[buf * _RPR + s], [lanes + 16 * a, loc])
                dst[jj, pl.ds(16 * a, 16)] = vals

    pltpu.sync_copy(urows_v, u_out.at[pl.ds(base, _BPW)])
    pltpu.sync_copy(irows_v, i_out.at[pl.ds(base, _BPW)])


def _make_gather():
    return pl.kernel(
        _gather_body,
        out_type=(
            jax.ShapeDtypeStruct((B, D), jnp.float32),
            jax.ShapeDtypeStruct((B, D), jnp.float32),
        ),
        mesh=plsc.VectorSubcoreMesh(
            core_axis_name="c", subcore_axis_name="s",
            num_cores=_NC, num_subcores=_NS,
        ),
        scratch_types=[
            pltpu.VMEM((_BPW,), jnp.int32),
            pltpu.VMEM((_BPW,), jnp.int32),
            pltpu.VMEM((_BPW, D), jnp.float32),
            pltpu.VMEM((_BPW, D), jnp.float32),
            pltpu.VMEM((_NBUF * _RPR, D, 128), jnp.float32),
            pltpu.SemaphoreType.DMA((_NBUF,)),
        ],
        compiler_params=pltpu.CompilerParams(use_tc_tiling_on_sc=True,
                                             needs_layout_passes=False),
    )


_RB = 128


def _mlp_body(b3_ref, u_ref, v_ref, w0_ref, b0_ref, w1_ref, b1_ref,
              w2_ref, b2_ref, w3r_ref, out_ref, gmf_ref):
    dn = (((1,), (1,)), ((), ()))
    hp = lax.Precision.DEFAULT
    i = pl.program_id(0)

    @pl.when(i == 0)
    def _():
        p = u_ref[...] * v_ref[...]
        gmf_ref[...] = lax.dot_general(jnp.ones((1, D), jnp.float32), p, dn,
                                       precision=hp)

    u = u_ref[pl.ds(i * _RB, _RB), :]
    v = v_ref[pl.ds(i * _RB, _RB), :]
    w0 = w0_ref[...]
    h = (lax.dot_general(u, w0[:, :D], dn, precision=hp)
         + lax.dot_general(v, w0[:, D:], dn, precision=hp) + b0_ref[...])
    h = jnp.maximum(h, 0.0)
    h = jnp.maximum(
        lax.dot_general(h, w1_ref[...], dn, precision=hp) + b1_ref[...], 0.0)
    h = jnp.maximum(
        lax.dot_general(h, w2_ref[...], dn, precision=hp) + b2_ref[...], 0.0)
    m_col = lax.dot_general(h, w3r_ref[...], dn, precision=hp)
    out_ref[...] = jax.nn.sigmoid(m_col + gmf_ref[...] + b3_ref[0])


def _mlp_call(u_emb, i_emb, W0, b0, W1, b1, W2, b2, W3, b3):
    full = lambda r, c: pl.BlockSpec((r, c), lambda i: (0, 0))
    return pl.pallas_call(
        _mlp_body,
        grid=(B // _RB,),
        out_shape=jax.ShapeDtypeStruct((B, B), jnp.float32),
        in_specs=[pl.BlockSpec(memory_space=pltpu.SMEM),
                  full(B, D), full(B, D),
                  full(256, 2 * D), full(1, 256),
                  full(128, 256), full(1, 128),
                  full(64, 128), full(1, 64),
                  full(1, D)],
        out_specs=pl.BlockSpec((_RB, B), lambda i: (i, 0)),
        scratch_shapes=[pltpu.VMEM((1, B), jnp.float32)],
    )(b3, u_emb, i_emb, W0, b0, W1, b1, W2, b2, W3)


def kernel(user_indices, item_indices, user_table, item_table,
           W0, b0, W1, b1, W2, b2, W3, b3):
    u_emb, i_emb = _make_gather()(user_indices.astype(jnp.int32),
                           item_indices.astype(jnp.int32),
                           user_table.T, item_table.T)
    return _mlp_call(u_emb, i_emb,
                     W0, b0.reshape(1, -1), W1, b1.reshape(1, -1),
                     W2, b2.reshape(1, -1), W3, b3)


# speedup vs baseline: 1.0478x; 1.0478x over previous
import jax
import jax.numpy as jnp
from jax import lax
from jax.experimental import pallas as pl
from jax.experimental.pallas import tpu as pltpu
from jax.experimental.pallas import tpu_sc as plsc

B = 1024
D = 64
_NC = 2
_NS = 16
_NW = _NC * _NS
_BPW = B // _NW


_RPR = 4
_NBUF = 3
_NROUND = 2 * _BPW // _RPR


def _gather_body(u_idx, i_idx, u_tabT, i_tabT, u_out, i_out,
                 uidx_v, iidx_v, urows_v, irows_v, slabs, sems, wbsem):
    wid = lax.axis_index("s") * _NC + lax.axis_index("c")
    base = wid * _BPW
    pltpu.sync_copy(u_idx.at[pl.ds(base, _BPW)], uidx_v)
    pltpu.sync_copy(i_idx.at[pl.ds(base, _BPW)], iidx_v)

    idx_vecs = []
    for k in range(_BPW // 16):
        idx_vecs.append(uidx_v[pl.ds(16 * k, 16)])
    for k in range(_BPW // 16):
        idx_vecs.append(iidx_v[pl.ds(16 * k, 16)])

    def row_scalar(j):
        return idx_vecs[j // 16][j % 16]

    def fire(rnd, buf):
        tab = u_tabT if rnd < _NROUND // 2 else i_tabT
        cps = []
        for s in range(_RPR):
            r = row_scalar(rnd * _RPR + s)
            col0 = pl.multiple_of((r >> 7) << 7, 128)
            cps.append(pltpu.async_copy(
                tab.at[:, pl.ds(col0, 128)], slabs.at[buf * _RPR + s],
                sems.at[buf]))
        return cps

    lanes = lax.iota(jnp.int32, 16)
    u_wb = None
    pend = {0: fire(0, 0), 1: fire(1, 1)}
    for rnd in range(_NROUND):
        buf = rnd % _NBUF
        if rnd + 2 < _NROUND:
            pend[(rnd + 2) % _NBUF] = fire(rnd + 2, (rnd + 2) % _NBUF)
        for cp in pend.pop(buf):
            cp.wait()
        for s in range(_RPR):
            j = rnd * _RPR + s
            r = row_scalar(j)
            loc = jnp.broadcast_to(r & 127, (16,))
            dst = urows_v if j < _BPW else irows_v
            jj = j % _BPW
            for a in range(D // 16):
                vals = plsc.load_gather(
                    slabs.at[buf * _RPR + s], [lanes + 16 * a, loc])
                dst[jj, pl.ds(16 * a, 16)] = vals
        if rnd == _NROUND // 2 - 1:
            u_wb = pltpu.async_copy(
                urows_v, u_out.at[pl.ds(base, _BPW)], wbsem)

    u_wb.wait()
    pltpu.sync_copy(irows_v, i_out.at[pl.ds(base, _BPW)])


def _make_gather():
    return pl.kernel(
        _gather_body,
        out_type=(
            jax.ShapeDtypeStruct((B, D), jnp.float32),
            jax.ShapeDtypeStruct((B, D), jnp.float32),
        ),
        mesh=plsc.VectorSubcoreMesh(
            core_axis_name="c", subcore_axis_name="s",
            num_cores=_NC, num_subcores=_NS,
        ),
        scratch_types=[
            pltpu.VMEM((_BPW,), jnp.int32),
            pltpu.VMEM((_BPW,), jnp.int32),
            pltpu.VMEM((_BPW, D), jnp.float32),
            pltpu.VMEM((_BPW, D), jnp.float32),
            pltpu.VMEM((_NBUF * _RPR, D, 128), jnp.float32),
            pltpu.SemaphoreType.DMA((_NBUF,)),
            pltpu.SemaphoreType.DMA,
        ],
        compiler_params=pltpu.CompilerParams(use_tc_tiling_on_sc=True,
                                             needs_layout_passes=False),
    )


def _mlp_body(b3_ref, u_ref, v_ref, w0_ref, b0_ref, w1_ref, b1_ref,
              w2_ref, b2_ref, w3r_ref, out_ref):
    dn = (((1,), (1,)), ((), ()))
    hp = lax.Precision.DEFAULT
    u = u_ref[...]
    v = v_ref[...]
    p = u * v
    gmf_row = lax.dot_general(jnp.ones((1, D), jnp.float32), p, dn,
                              precision=hp)
    w0 = w0_ref[...]
    h = (lax.dot_general(u, w0[:, :D], dn, precision=hp)
         + lax.dot_general(v, w0[:, D:], dn, precision=hp) + b0_ref[...])
    h = jnp.maximum(h, 0.0)
    h = jnp.maximum(
        lax.dot_general(h, w1_ref[...], dn, precision=hp) + b1_ref[...], 0.0)
    h = jnp.maximum(
        lax.dot_general(h, w2_ref[...], dn, precision=hp) + b2_ref[...], 0.0)
    m_col = lax.dot_general(h, w3r_ref[...], dn, precision=hp)
    out_ref[...] = jax.nn.sigmoid(m_col + gmf_row + b3_ref[0])


def _mlp_call(u_emb, i_emb, W0, b0, W1, b1, W2, b2, W3, b3):
    w3r = W3
    return pl.pallas_call(
        _mlp_body,
        out_shape=jax.ShapeDtypeStruct((B, B), jnp.float32),
        in_specs=[pl.BlockSpec(memory_space=pltpu.SMEM)] + [
            pl.BlockSpec(memory_space=pltpu.VMEM)] * 9,
        out_specs=pl.BlockSpec(memory_space=pltpu.VMEM),
    )(b3, u_emb, i_emb, W0, b0, W1, b1, W2, b2, w3r)


def kernel(user_indices, item_indices, user_table, item_table,
           W0, b0, W1, b1, W2, b2, W3, b3):
    u_emb, i_emb = _make_gather()(user_indices.astype(jnp.int32),
                           item_indices.astype(jnp.int32),
                           user_table.T, item_table.T)
    return _mlp_call(u_emb, i_emb,
                     W0, b0.reshape(1, -1), W1, b1.reshape(1, -1),
                     W2, b2.reshape(1, -1), W3, b3)


# speedup vs baseline: 1.0921x; 1.0423x over previous
import jax
import jax.numpy as jnp
from jax import lax
from jax.experimental import pallas as pl
from jax.experimental.pallas import tpu as pltpu
from jax.experimental.pallas import tpu_sc as plsc

B = 1024
D = 64
_NC = 2
_NS = 16
_NW = _NC * _NS
_BPW = B // _NW


_RPR = 4
_NBUF = 3
_NROUND = 2 * _BPW // _RPR


def _gather_body(u_idx, i_idx, u_tabT, i_tabT, u_out, i_out,
                 idxall_v, rows_v, slabs, sems, wbsem):
    wid = lax.axis_index("s") * _NC + lax.axis_index("c")
    base = wid * _BPW
    pltpu.sync_copy(u_idx.at[pl.ds(base, _BPW)], idxall_v.at[pl.ds(0, _BPW)])
    pltpu.sync_copy(i_idx.at[pl.ds(base, _BPW)],
                    idxall_v.at[pl.ds(_BPW, _BPW)])

    lanes = lax.iota(jnp.int32, 16)

    def row_scalar(j):
        return jnp.max(plsc.load_gather(
            idxall_v, [jnp.broadcast_to(j, (16,))]))

    def fire(rnd, buf):
        def issue(tab, jb):
            for s in range(_RPR):
                r = row_scalar(jb + s)
                col0 = pl.multiple_of((r >> 7) << 7, 128)
                pltpu.async_copy(tab.at[:, pl.ds(col0, 128)],
                                 slabs.at[buf * _RPR + s], sems.at[buf])

        @pl.when(rnd < _NROUND // 2)
        def _():
            issue(u_tabT, rnd * _RPR)

        @pl.when(rnd >= _NROUND // 2)
        def _():
            issue(i_tabT, rnd * _RPR)

    def drain(buf):
        for s in range(_RPR):
            pltpu.make_async_copy(u_tabT.at[:, pl.ds(0, 128)],
                                  slabs.at[buf * _RPR + s],
                                  sems.at[buf]).wait()

    def extract(rnd, buf):
        for s in range(_RPR):
            j = rnd * _RPR + s
            r = row_scalar(j)
            loc = jnp.broadcast_to(r & 127, (16,))
            jv = jnp.broadcast_to(j, (16,))
            for a in range(D // 16):
                vals = plsc.load_gather(
                    slabs.at[buf * _RPR + s], [lanes + 16 * a, loc])
                plsc.store_scatter(rows_v, [jv, lanes + 16 * a], vals)

    fire(jnp.int32(0), 0)
    fire(jnp.int32(1), 1)

    def body(g, carry):
        for b in range(_NBUF):
            rnd = _NBUF * g + b

            @pl.when(rnd + 2 < _NROUND)
            def _():
                fire(rnd + 2, (b + 2) % _NBUF)

            drain(b)
            extract(rnd, b)

            @pl.when(rnd == _NROUND // 2 - 1)
            def _():
                pltpu.async_copy(rows_v.at[pl.ds(0, _BPW)],
                                 u_out.at[pl.ds(base, _BPW)], wbsem)
        return carry

    lax.fori_loop(0, (_NROUND - 1) // _NBUF, body, jnp.int32(0))
    tail = _NROUND - 1
    drain(tail % _NBUF)
    extract(jnp.int32(tail), tail % _NBUF)

    pltpu.make_async_copy(rows_v.at[pl.ds(0, _BPW)],
                          u_out.at[pl.ds(base, _BPW)], wbsem).wait()
    pltpu.sync_copy(rows_v.at[pl.ds(_BPW, _BPW)],
                    i_out.at[pl.ds(base, _BPW)])


def _make_gather():
    return pl.kernel(
        _gather_body,
        out_type=(
            jax.ShapeDtypeStruct((B, D), jnp.float32),
            jax.ShapeDtypeStruct((B, D), jnp.float32),
        ),
        mesh=plsc.VectorSubcoreMesh(
            core_axis_name="c", subcore_axis_name="s",
            num_cores=_NC, num_subcores=_NS,
        ),
        scratch_types=[
            pltpu.VMEM((2 * _BPW,), jnp.int32),
            pltpu.VMEM((2 * _BPW, D), jnp.float32),
            pltpu.VMEM((_NBUF * _RPR, D, 128), jnp.float32),
            pltpu.SemaphoreType.DMA((_NBUF,)),
            pltpu.SemaphoreType.DMA,
        ],
        compiler_params=pltpu.CompilerParams(use_tc_tiling_on_sc=True,
                                             needs_layout_passes=False),
    )


def _mlp_body(b3_ref, u_ref, v_ref, w0_ref, b0_ref, w1_ref, b1_ref,
              w2_ref, b2_ref, w3r_ref, out_ref):
    dn = (((1,), (1,)), ((), ()))
    hp = lax.Precision.DEFAULT
    u = u_ref[...]
    v = v_ref[...]
    p = u * v
    gmf_row = lax.dot_general(jnp.ones((1, D), jnp.float32), p, dn,
                              precision=hp)
    w0 = w0_ref[...]
    h = (lax.dot_general(u, w0[:, :D], dn, precision=hp)
         + lax.dot_general(v, w0[:, D:], dn, precision=hp) + b0_ref[...])
    h = jnp.maximum(h, 0.0)
    h = jnp.maximum(
        lax.dot_general(h, w1_ref[...], dn, precision=hp) + b1_ref[...], 0.0)
    h = jnp.maximum(
        lax.dot_general(h, w2_ref[...], dn, precision=hp) + b2_ref[...], 0.0)
    m_col = lax.dot_general(h, w3r_ref[...], dn, precision=hp)
    out_ref[...] = jax.nn.sigmoid(m_col + gmf_row + b3_ref[0])


def _mlp_call(u_emb, i_emb, W0, b0, W1, b1, W2, b2, W3, b3):
    w3r = W3
    return pl.pallas_call(
        _mlp_body,
        out_shape=jax.ShapeDtypeStruct((B, B), jnp.float32),
        in_specs=[pl.BlockSpec(memory_space=pltpu.SMEM)] + [
            pl.BlockSpec(memory_space=pltpu.VMEM)] * 9,
        out_specs=pl.BlockSpec(memory_space=pltpu.VMEM),
    )(b3, u_emb, i_emb, W0, b0, W1, b1, W2, b2, w3r)


def kernel(user_indices, item_indices, user_table, item_table,
           W0, b0, W1, b1, W2, b2, W3, b3):
    u_emb, i_emb = _make_gather()(user_indices.astype(jnp.int32),
                           item_indices.astype(jnp.int32),
                           user_table.T, item_table.T)
    return _mlp_call(u_emb, i_emb,
                     W0, b0.reshape(1, -1), W1, b1.reshape(1, -1),
                     W2, b2.reshape(1, -1), W3, b3)
